# baseline (speedup 1.0000x reference)
import functools

import jax
import jax.numpy as jnp
from jax import lax
from jax.experimental import pallas as pl
from jax.experimental.pallas import tpu as pltpu
from jax.experimental.pallas import tpu_sc as plsc

_NUM_CORES = 2
_CHUNK_ROWS = 1024


def kernel(x, emb):
    batch = x.shape[0]
    n_rows, dim = emb.shape
    rows_per_c = n_rows // _NUM_CORES
    n_chunks = rows_per_c // _CHUNK_ROWS

    mesh = plsc.ScalarSubcoreMesh(axis_name="c", num_cores=_NUM_CORES)

    @functools.partial(
        pl.kernel,
        mesh=mesh,
        out_type=jax.ShapeDtypeStruct((batch, n_rows, dim), jnp.float32),
        scratch_types=[
            pltpu.VMEM_SHARED((_CHUNK_ROWS, dim), jnp.float32),
            pltpu.VMEM_SHARED((_CHUNK_ROWS, dim), jnp.float32),
            pltpu.SemaphoreType.DMA,
            pltpu.SemaphoreType.DMA,
        ],
    )
    def k(emb_hbm, out_hbm, buf0, buf1, sem_r, sem_w):
        cid = lax.axis_index("c")
        base = cid * rows_per_c
        bufs = (buf0, buf1)

        def read(i):
            r0 = base + i * _CHUNK_ROWS
            return pltpu.async_copy(
                emb_hbm.at[pl.ds(r0, _CHUNK_ROWS), :], bufs[i % 2], sem_r
            )

        def writes(i):
            r0 = base + i * _CHUNK_ROWS
            return [
                pltpu.async_copy(
                    bufs[i % 2], out_hbm.at[b, pl.ds(r0, _CHUNK_ROWS), :], sem_w
                )
                for b in range(batch)
            ]

        pending = [None] * n_chunks
        reads = [None] * n_chunks
        reads[0] = read(0)
        for i in range(n_chunks):
            reads[i].wait()
            pending[i] = writes(i)
            if i + 1 < n_chunks:
                if i - 1 >= 0:
                    for c in pending[i - 1]:
                        c.wait()
                    pending[i - 1] = None
                reads[i + 1] = read(i + 1)
        for ws in pending:
            if ws is not None:
                for c in ws:
                    c.wait()

    return k(emb)
